# baseline (speedup 1.0000x reference)
import dataclasses
import functools

import jax
import jax.numpy as jnp
from jax import lax
from jax.experimental import pallas as pl
from jax.experimental.pallas import tpu as pltpu
from jax.experimental.pallas import tpu_sc as plsc

N = 10000
E = 160000
R = 12
NB = 8
HID = 768
EHID = 384
BATCH = 16
ELEN = 32
TLEN = 64
NLAYER = 12
NBLOCK = 2
NHEAD = 12

NPOS = BATCH * ELEN
NSC = 2
NTILE = 16
SLOTS_PER_SC = NPOS // NSC
ROWS_PER_SC = SLOTS_PER_SC * R
ACC_ROWS = ROWS_PER_SC + 16
SM_PAD = 10240
EDGES_PER_TILE = E // NTILE
ECH = 400
GCH = 64
LIST_PAD = EDGES_PER_TILE + GCH
KPT = NPOS // NTILE
ZROWS = 16

_MESH = dict(core_axis_name="c", subcore_axis_name="s")


def _sc_params():
    cp = pltpu.CompilerParams()
    if "needs_layout_passes" in pltpu.CompilerParams.__dataclass_fields__:
        cp = dataclasses.replace(cp, needs_layout_passes=False)
    return cp


def _first_idx_kernel(eid_col, eid_row):
    def body(a_ref, b_ref, o_ref):
        eq = a_ref[...] == b_ref[...]
        jj = lax.broadcasted_iota(jnp.int32, (NPOS, NPOS), 1)
        o_ref[...] = jnp.min(jnp.where(eq, jj, NPOS), axis=1, keepdims=True)
    return pl.pallas_call(
        body,
        out_shape=jax.ShapeDtypeStruct((NPOS, 1), jnp.int32),
    )(eid_col, eid_row)


NWORK = NSC * NTILE
SPT = NPOS // NWORK
AROWS = SPT * R
ADUMMY = AROWS
ACCP = AROWS + 8
LIST_CAP = 8192
EPW = E // NWORK
EPWP = EPW + 16


def _sc_scan(eid, fidx, edge_index, et):
    mesh = plsc.VectorSubcoreMesh(**_MESH)

    @functools.partial(
        pl.kernel,
        out_type=(
            jax.ShapeDtypeStruct((NWORK, EPWP), jnp.int32),
            jax.ShapeDtypeStruct((NWORK, 16), jnp.int32),
        ),
        mesh=mesh,
        compiler_params=_sc_params(),
        scratch_types=[
            pltpu.VMEM((SM_PAD,), jnp.int32),
            pltpu.VMEM((NPOS,), jnp.int32),
            pltpu.VMEM((NPOS,), jnp.int32),
            pltpu.VMEM((EPW,), jnp.int32),
            pltpu.VMEM((EPW,), jnp.int32),
            pltpu.VMEM((EPW,), jnp.int32),
            pltpu.VMEM((EPWP,), jnp.int32),
            pltpu.VMEM((16,), jnp.int32),
        ],
    )
    def k(eid_hbm, fidx_hbm, ei_hbm, et_hbm, list_hbm, n_hbm,
          sm_vm, eid_vm, val_vm, esrc_vm, edst_vm, etyp_vm, plist_vm, n_vm):
        c = lax.axis_index("c")
        s = lax.axis_index("s")
        w = s * NSC + c

        @pl.loop(0, SM_PAD, step=16)
        def _(i):
            sm_vm[pl.ds(i, 16)] = jnp.full((16,), -1, jnp.int32)

        pltpu.sync_copy(eid_hbm, eid_vm)
        pltpu.sync_copy(fidx_hbm, val_vm)

        @pl.loop(0, NPOS, step=16)
        def _(j):
            plsc.store_scatter(sm_vm, [eid_vm[pl.ds(j, 16)]],
                               val_vm[pl.ds(j, 16)])

        eb = w * EPW
        pltpu.sync_copy(ei_hbm.at[pl.ds(eb, EPW)], esrc_vm)
        pltpu.sync_copy(ei_hbm.at[pl.ds(E + eb, EPW)], edst_vm)
        pltpu.sync_copy(et_hbm.at[pl.ds(eb, EPW)], etyp_vm)

        def vec_body(jj, off):
            j = jj * 16
            sv = esrc_vm[pl.ds(j, 16)]
            dv = edst_vm[pl.ds(j, 16)]
            tv = etyp_vm[pl.ds(j, 16)]
            slot = plsc.load_gather(sm_vm, [dv])
            msk = slot >= 0
            packed = (sv << 13) | jnp.where(msk, slot << 4, 0) | tv
            plsc.store_compressed(plist_vm.at[pl.ds(off, 16)], packed,
                                  mask=msk)
            return off + jnp.sum(msk.astype(jnp.int32))

        n_sel = lax.fori_loop(0, EPW // 16, vec_body, jnp.int32(0))

        n_vm[pl.ds(0, 16)] = jnp.where(
            lax.iota(jnp.int32, 16) == 0, n_sel, 0)
        pltpu.sync_copy(plist_vm, list_hbm.at[w])
        pltpu.sync_copy(n_vm, n_hbm.at[w])

    return k(eid, fidx, edge_index.reshape(2 * E), et)


def _sc_accumulate(lists, counts, eid, x, zrows):
    mesh = plsc.VectorSubcoreMesh(**_MESH)

    @functools.partial(
        pl.kernel,
        out_type=(
            jax.ShapeDtypeStruct((NPOS * R, EHID), jnp.float32),
            jax.ShapeDtypeStruct((NPOS * R,), jnp.float32),
            jax.ShapeDtypeStruct((NPOS, EHID), jnp.float32),
        ),
        mesh=mesh,
        compiler_params=_sc_params(),
        scratch_types=[
            pltpu.VMEM((ACCP, EHID), jnp.float32),
            pltpu.VMEM((GCH, EHID), jnp.float32),
            pltpu.VMEM((GCH,), jnp.int32),
            pltpu.VMEM((GCH + 16,), jnp.int32),
            pltpu.VMEM((ACCP,), jnp.float32),
            pltpu.VMEM((NPOS,), jnp.int32),
            pltpu.VMEM((EPWP,), jnp.int32),
            pltpu.VMEM((16,), jnp.int32),
            pltpu.VMEM((LIST_CAP + EPWP,), jnp.int32),
            pltpu.SemaphoreType.DMA,
        ],
    )
    def k(list_hbm, n_hbm, eid_hbm, x_hbm, zr_hbm,
          m_hbm, cnt_hbm, xs_hbm,
          acc_vm, rows_vm, gstage_vm, astage_vm, cnt_vm,
          eid_vm, src_vm, n_vm, plist_vm, sem):
        c = lax.axis_index("c")
        s = lax.axis_index("s")
        w = s * NSC + c
        lo = w * SPT

        pltpu.sync_copy(zr_hbm, acc_vm)

        @pl.loop(0, ACCP, step=16)
        def _(i):
            cnt_vm[pl.ds(i, 16)] = jnp.zeros((16,), jnp.float32)

        pltpu.sync_copy(eid_hbm, eid_vm)
        pltpu.async_copy(x_hbm.at[eid_vm.at[pl.ds(lo, SPT)]],
                         rows_vm.at[pl.ds(0, SPT)], sem).wait()
        pltpu.sync_copy(rows_vm.at[pl.ds(0, SPT)], xs_hbm.at[pl.ds(lo, SPT)])

        def drain(n_items):
            nch = (n_items + (GCH - 1)) // GCH

            def gs_body(i, carry):
                gbase = i * GCH
                for kk in range(GCH // 16):
                    pos = gbase + kk * 16
                    lane = pos + lax.iota(jnp.int32, 16)
                    valid = lane < n_items
                    pk = plist_vm[pl.ds(pos, 16)]
                    gstage_vm[pl.ds(kk * 16, 16)] = jnp.where(
                        valid, lax.shift_right_logical(pk, 8), 0)
                    astage_vm[pl.ds(kk * 16, 16)] = jnp.where(
                        valid, pk & 255, ADUMMY)
                    plsc.addupdate_scatter(
                        cnt_vm, [jnp.where(valid, pk & 255, ADUMMY)],
                        jnp.ones((16,), jnp.float32), mask=valid)
                pltpu.async_copy(x_hbm.at[gstage_vm], rows_vm, sem).wait()

                @pl.loop(0, GCH)
                def _(rr):
                    aidx = astage_vm[pl.ds(rr, 16)][0]

                    @pl.loop(0, EHID, step=16)
                    def _(kk):
                        v = rows_vm[rr, pl.ds(kk, 16)]
                        plsc.addupdate(acc_vm.at[aidx, pl.ds(kk, 16)], v)

                return carry

            lax.fori_loop(0, nch, gs_body, jnp.int32(0))

        SHORT = 512

        def take_list(v, off):
            pltpu.sync_copy(n_hbm.at[v], n_vm)
            n_v = n_vm[pl.ds(0, 16)][0]

            @pl.when(n_v <= SHORT)
            def _():
                pltpu.sync_copy(list_hbm.at[v, pl.ds(0, SHORT)],
                                src_vm.at[pl.ds(0, SHORT)])

            @pl.when(n_v > SHORT)
            def _():
                pltpu.sync_copy(list_hbm.at[v], src_vm)

            def vec_body(jj, off2):
                j = jj * 16
                pk = src_vm[pl.ds(j, 16)]
                valid = (j + lax.iota(jnp.int32, 16)) < n_v
                slot = lax.shift_right_logical(pk, 4) & 511
                sl = slot - lo
                msk = valid & (sl >= 0) & (sl < SPT)
                aidx = jnp.where(msk, sl * R + (pk & 15), 0)
                repk = (lax.shift_right_logical(pk, 13) << 8) | aidx
                plsc.store_compressed(plist_vm.at[pl.ds(off2, 16)], repk,
                                      mask=msk)
                return off2 + jnp.sum(msk.astype(jnp.int32))

            nv16 = (n_v + 15) // 16
            off = lax.fori_loop(0, nv16, vec_body, off)

            def do_drain(o):
                drain(o)
                return jnp.int32(0)

            return lax.cond(off >= LIST_CAP, do_drain, lambda o: o, off)

        n_left = lax.fori_loop(0, NWORK, take_list, jnp.int32(0))
        drain(n_left)

        pltpu.sync_copy(acc_vm.at[pl.ds(0, AROWS)],
                        m_hbm.at[pl.ds(w * AROWS, AROWS)])
        pltpu.sync_copy(cnt_vm.at[pl.ds(0, AROWS)],
                        cnt_hbm.at[pl.ds(w * AROWS, AROWS)])

    return k(lists, counts, eid, x, zrows)


def _sc_aggregate(eid, fidx, edge_index, et, x, zrows):
    lists, counts = _sc_scan(eid, fidx, edge_index, et)
    return _sc_accumulate(lists, counts, eid, x, zrows)


def _entity_kernel(m2, cntp, xs, comp, bases, root, rgcn_bias,
                   ep1_w1, ep1_b1, ep1_w2, ep1_b2, ep2_w, ep2_b):
    def body(m_ref, c_ref, xs_ref, comp_ref, bases_ref, root_ref, b0_ref,
             w1_ref, b1_ref, w2_ref, b2_ref, w3_ref, b3_ref, out_ref):
        inv = 1.0 / jnp.maximum(c_ref[...], 1.0)
        agg = jnp.zeros((NPOS, EHID), jnp.float32)
        for r in range(R):
            wr = jnp.zeros((EHID, EHID), jnp.float32)
            for b in range(NB):
                wr = wr + comp_ref[r, b] * bases_ref[b]
            mr = m_ref[:, r * EHID:(r + 1) * EHID] * inv[:, r:r + 1]
            agg = agg + jnp.dot(mr, wr, preferred_element_type=jnp.float32)
        x = xs_ref[...]
        ent0 = (agg + jnp.dot(x, root_ref[...],
                              preferred_element_type=jnp.float32)
                + b0_ref[...] + x)
        h = jnp.maximum(
            jnp.dot(ent0, w1_ref[...], preferred_element_type=jnp.float32)
            + b1_ref[...], 0.0)
        ent1 = (jnp.dot(h, w2_ref[...], preferred_element_type=jnp.float32)
                + b2_ref[...] + ent0)
        out_ref[...] = (jnp.dot(ent1, w3_ref[...],
                                preferred_element_type=jnp.float32)
                        + b3_ref[...])

    return pl.pallas_call(
        body,
        out_shape=jax.ShapeDtypeStruct((NPOS, HID), jnp.float32),
        in_specs=[pl.BlockSpec((NPOS, R * EHID), lambda: (0, 0)),
                 pl.BlockSpec((NPOS, R), lambda: (0, 0)),
                 pl.BlockSpec((NPOS, EHID), lambda: (0, 0)),
                 pl.BlockSpec(memory_space=pltpu.SMEM),
                 pl.BlockSpec((NB, EHID, EHID), lambda: (0, 0, 0)),
                 pl.BlockSpec((EHID, EHID), lambda: (0, 0)),
                 pl.BlockSpec((EHID,), lambda: (0,)),
                 pl.BlockSpec((EHID, EHID // 2), lambda: (0, 0)),
                 pl.BlockSpec((EHID // 2,), lambda: (0,)),
                 pl.BlockSpec((EHID // 2, EHID), lambda: (0, 0)),
                 pl.BlockSpec((EHID,), lambda: (0,)),
                 pl.BlockSpec((EHID, HID), lambda: (0, 0)),
                 pl.BlockSpec((HID,), lambda: (0,))],
        out_specs=pl.BlockSpec((NPOS, HID), lambda: (0, 0)),
    )(m2, cntp, xs, comp, bases, root, rgcn_bias,
      ep1_w1, ep1_b1, ep1_w2, ep1_b2, ep2_w, ep2_b)


def _sc_gather(ent, sidx):
    mesh = plsc.VectorSubcoreMesh(**_MESH)
    per_w = NPOS // (NSC * NTILE)

    @functools.partial(
        pl.kernel,
        out_type=jax.ShapeDtypeStruct((NPOS, HID), jnp.float32),
        mesh=mesh,
        scratch_types=[
            pltpu.VMEM((per_w,), jnp.int32),
            pltpu.VMEM((per_w, HID), jnp.float32),
            pltpu.SemaphoreType.DMA,
        ],
    )
    def k(ent_hbm, sidx_hbm, out_hbm, idx_vm, rows_vm, sem):
        c = lax.axis_index("c")
        s = lax.axis_index("s")
        base = (s * NSC + c) * per_w
        pltpu.sync_copy(sidx_hbm.at[pl.ds(base, per_w)], idx_vm)
        pltpu.async_copy(ent_hbm.at[idx_vm], rows_vm, sem).wait()
        pltpu.sync_copy(rows_vm, out_hbm.at[pl.ds(base, per_w)])

    return k(ent, sidx)


def _token_kernel(tok, w1, b1, w2, b2, w3, b3):
    def body(t_ref, w1_ref, b1_ref, w2_ref, b2_ref, w3_ref, b3_ref, o_ref):
        t0 = t_ref[...]
        h = jnp.maximum(
            jnp.dot(t0, w1_ref[...], preferred_element_type=jnp.float32)
            + b1_ref[...], 0.0)
        t1 = (jnp.dot(h, w2_ref[...], preferred_element_type=jnp.float32)
              + b2_ref[...] + t0)
        o_ref[...] = (jnp.dot(t1, w3_ref[...],
                              preferred_element_type=jnp.float32)
                      + b3_ref[...])

    return pl.pallas_call(
        body,
        out_shape=jax.ShapeDtypeStruct((BATCH * TLEN, HID), jnp.float32),
    )(tok, w1, b1, w2, b2, w3, b3)


def _attn_kernel(t2, ee, ca_w, w1, b1, w2, b2):
    def body(t_ref, e_ref, ca_ref, w1_ref, b1_ref, w2_ref, b2_ref, o_ref):
        tb = t_ref[0]
        eb = e_ref[0]
        q = jnp.dot(tb, ca_ref[...], preferred_element_type=jnp.float32)
        attn = lax.dot_general(q, eb, (((1,), (1,)), ((), ())),
                               preferred_element_type=jnp.float32) / HID
        mx = jnp.max(attn, axis=0, keepdims=True)
        ex = jnp.exp(attn - mx)
        sm = ex / jnp.sum(ex, axis=0, keepdims=True)
        p0 = lax.dot_general(sm, tb, (((0,), (0,)), ((), ())),
                             preferred_element_type=jnp.float32) + eb
        h = jnp.maximum(
            jnp.dot(p0, w1_ref[...], preferred_element_type=jnp.float32)
            + b1_ref[...], 0.0)
        o_ref[0] = (jnp.dot(h, w2_ref[...], preferred_element_type=jnp.float32)
                    + b2_ref[...] + p0)

    return pl.pallas_call(
        body,
        grid=(BATCH,),
        in_specs=[
            pl.BlockSpec((1, TLEN, HID), lambda i: (i, 0, 0)),
            pl.BlockSpec((1, ELEN, HID), lambda i: (i, 0, 0)),
            pl.BlockSpec((HID, HID), lambda i: (0, 0)),
            pl.BlockSpec((HID, HID // 2), lambda i: (0, 0)),
            pl.BlockSpec((HID // 2,), lambda i: (0,)),
            pl.BlockSpec((HID // 2, HID), lambda i: (0, 0)),
            pl.BlockSpec((HID,), lambda i: (0,)),
        ],
        out_specs=pl.BlockSpec((1, ELEN, HID), lambda i: (i, 0, 0)),
        out_shape=jax.ShapeDtypeStruct((BATCH, ELEN, HID), jnp.float32),
    )(t2, ee, ca_w, w1, b1, w2, b2)


def _pp2_kernel(x, w, b):
    OUTD = NLAYER * NBLOCK * HID
    CB = 1536

    def body(x_ref, w_ref, b_ref, o_ref):
        o_ref[...] = (jnp.dot(x_ref[...].astype(jnp.bfloat16), w_ref[...],
                              preferred_element_type=jnp.float32)
                      + b_ref[...])

    return pl.pallas_call(
        body,
        grid=(OUTD // CB,),
        in_specs=[
            pl.BlockSpec((NPOS, HID), lambda j: (0, 0)),
            pl.BlockSpec((HID, CB), lambda j: (0, j)),
            pl.BlockSpec((1, CB), lambda j: (0, j)),
        ],
        out_specs=pl.BlockSpec((NPOS, CB), lambda j: (0, j)),
        out_shape=jax.ShapeDtypeStruct((NPOS, OUTD), jnp.float32),
    )(x, w.astype(jnp.bfloat16), b.reshape(1, OUTD))


def kernel(entity_ids, token_embeds, edge_index, edge_type, node_embeds,
           comp, bases, root, rgcn_bias, ep1_w1, ep1_b1, ep1_w2, ep1_b2,
           ep2_w, ep2_b, tp1_w1, tp1_b1, tp1_w2, tp1_b2, tp2_w, tp2_b,
           ca_w, pp1_w1, pp1_b1, pp1_w2, pp1_b2, pp2_w, pp2_b):
    eid = entity_ids.reshape(NPOS).astype(jnp.int32)
    ei = edge_index.astype(jnp.int32)
    et = edge_type.astype(jnp.int32)

    fidx = _first_idx_kernel(eid.reshape(NPOS, 1),
                             eid.reshape(1, NPOS)).reshape(NPOS)

    zrows = jnp.zeros((ACCP, EHID), jnp.float32)
    m, cnt, xs = _sc_aggregate(eid, fidx, ei, et, node_embeds, zrows)

    m2 = m.reshape(NPOS, R * EHID)
    cnt2 = cnt.reshape(NPOS, R)

    ent = _entity_kernel(m2, cnt2, xs, comp, bases, root, rgcn_bias,
                         ep1_w1, ep1_b1, ep1_w2, ep1_b2, ep2_w, ep2_b)

    ee = _sc_gather(ent, fidx)

    t2 = _token_kernel(token_embeds.reshape(BATCH * TLEN, HID),
                       tp1_w1, tp1_b1, tp1_w2, tp1_b2, tp2_w, tp2_b)

    presid = _attn_kernel(t2.reshape(BATCH, TLEN, HID),
                          ee.reshape(BATCH, ELEN, HID),
                          ca_w, pp1_w1, pp1_b1, pp1_w2, pp1_b2)

    pm = _pp2_kernel(presid.reshape(NPOS, HID), pp2_w, pp2_b)

    out = pm.reshape(BATCH, ELEN, NLAYER, NBLOCK, NHEAD, HID // NHEAD)
    return jnp.transpose(out, (2, 3, 0, 4, 1, 5))


# speedup vs baseline: 1.0612x; 1.0612x over previous
import dataclasses
import functools

import jax
import jax.numpy as jnp
from jax import lax
from jax.experimental import pallas as pl
from jax.experimental.pallas import tpu as pltpu
from jax.experimental.pallas import tpu_sc as plsc

N = 10000
E = 160000
R = 12
NB = 8
HID = 768
EHID = 384
BATCH = 16
ELEN = 32
TLEN = 64
NLAYER = 12
NBLOCK = 2
NHEAD = 12

NPOS = BATCH * ELEN
NSC = 2
NTILE = 16
SLOTS_PER_SC = NPOS // NSC
ROWS_PER_SC = SLOTS_PER_SC * R
ACC_ROWS = ROWS_PER_SC + 16
SM_PAD = 10240
EDGES_PER_TILE = E // NTILE
ECH = 400
GCH = 64
LIST_PAD = EDGES_PER_TILE + GCH
KPT = NPOS // NTILE
ZROWS = 16

_MESH = dict(core_axis_name="c", subcore_axis_name="s")


def _sc_params():
    cp = pltpu.CompilerParams()
    if "needs_layout_passes" in pltpu.CompilerParams.__dataclass_fields__:
        cp = dataclasses.replace(cp, needs_layout_passes=False)
    return cp


def _first_idx_kernel(eid_col, eid_row):
    def body(a_ref, b_ref, o_ref):
        eq = a_ref[...] == b_ref[...]
        jj = lax.broadcasted_iota(jnp.int32, (NPOS, NPOS), 1)
        o_ref[...] = jnp.min(jnp.where(eq, jj, NPOS), axis=1, keepdims=True)
    return pl.pallas_call(
        body,
        out_shape=jax.ShapeDtypeStruct((NPOS, 1), jnp.int32),
    )(eid_col, eid_row)


NWORK = NSC * NTILE
SPT = NPOS // NWORK
AROWS = SPT * R
ADUMMY = AROWS
ACCP = AROWS + 8
LIST_CAP = 8192
EPW = E // NWORK
EPWP = EPW + 16


def _sc_scan(eid, fidx, edge_index, et):
    mesh = plsc.VectorSubcoreMesh(**_MESH)

    @functools.partial(
        pl.kernel,
        out_type=(
            jax.ShapeDtypeStruct((NWORK, EPWP), jnp.int32),
            jax.ShapeDtypeStruct((NWORK, 16), jnp.int32),
        ),
        mesh=mesh,
        compiler_params=_sc_params(),
        scratch_types=[
            pltpu.VMEM((SM_PAD,), jnp.int32),
            pltpu.VMEM((NPOS,), jnp.int32),
            pltpu.VMEM((NPOS,), jnp.int32),
            pltpu.VMEM((EPW,), jnp.int32),
            pltpu.VMEM((EPW,), jnp.int32),
            pltpu.VMEM((EPW,), jnp.int32),
            pltpu.VMEM((EPWP,), jnp.int32),
            pltpu.VMEM((16,), jnp.int32),
        ],
    )
    def k(eid_hbm, fidx_hbm, ei_hbm, et_hbm, list_hbm, n_hbm,
          sm_vm, eid_vm, val_vm, esrc_vm, edst_vm, etyp_vm, plist_vm, n_vm):
        c = lax.axis_index("c")
        s = lax.axis_index("s")
        w = s * NSC + c

        @pl.loop(0, SM_PAD, step=16)
        def _(i):
            sm_vm[pl.ds(i, 16)] = jnp.full((16,), -1, jnp.int32)

        pltpu.sync_copy(eid_hbm, eid_vm)
        pltpu.sync_copy(fidx_hbm, val_vm)

        @pl.loop(0, NPOS, step=16)
        def _(j):
            plsc.store_scatter(sm_vm, [eid_vm[pl.ds(j, 16)]],
                               val_vm[pl.ds(j, 16)])

        eb = w * EPW
        pltpu.sync_copy(ei_hbm.at[pl.ds(eb, EPW)], esrc_vm)
        pltpu.sync_copy(ei_hbm.at[pl.ds(E + eb, EPW)], edst_vm)
        pltpu.sync_copy(et_hbm.at[pl.ds(eb, EPW)], etyp_vm)

        def vec_body(jj, off):
            j = jj * 16
            sv = esrc_vm[pl.ds(j, 16)]
            dv = edst_vm[pl.ds(j, 16)]
            tv = etyp_vm[pl.ds(j, 16)]
            slot = plsc.load_gather(sm_vm, [dv])
            msk = slot >= 0
            packed = (sv << 13) | jnp.where(msk, slot << 4, 0) | tv
            plsc.store_compressed(plist_vm.at[pl.ds(off, 16)], packed,
                                  mask=msk)
            return off + jnp.sum(msk.astype(jnp.int32))

        n_sel = lax.fori_loop(0, EPW // 16, vec_body, jnp.int32(0))

        n_vm[pl.ds(0, 16)] = jnp.where(
            lax.iota(jnp.int32, 16) == 0, n_sel, 0)
        pltpu.sync_copy(plist_vm, list_hbm.at[w])
        pltpu.sync_copy(n_vm, n_hbm.at[w])

    return k(eid, fidx, edge_index.reshape(2 * E), et)


def _sc_accumulate(lists, counts, eid, x, zrows):
    mesh = plsc.VectorSubcoreMesh(**_MESH)

    @functools.partial(
        pl.kernel,
        out_type=(
            jax.ShapeDtypeStruct((NPOS * R, EHID), jnp.float32),
            jax.ShapeDtypeStruct((NPOS * R,), jnp.float32),
            jax.ShapeDtypeStruct((NPOS, EHID), jnp.float32),
        ),
        mesh=mesh,
        compiler_params=_sc_params(),
        scratch_types=[
            pltpu.VMEM((ACCP, EHID), jnp.float32),
            pltpu.VMEM((GCH, EHID), jnp.float32),
            pltpu.VMEM((GCH,), jnp.int32),
            pltpu.VMEM((GCH + 16,), jnp.int32),
            pltpu.VMEM((ACCP,), jnp.float32),
            pltpu.VMEM((NPOS,), jnp.int32),
            pltpu.VMEM((EPWP,), jnp.int32),
            pltpu.VMEM((512,), jnp.int32),
            pltpu.VMEM((512,), jnp.int32),
            pltpu.VMEM((NWORK, 16), jnp.int32),
            pltpu.VMEM((LIST_CAP + EPWP,), jnp.int32),
            pltpu.SemaphoreType.DMA,
            pltpu.SemaphoreType.DMA,
            pltpu.SemaphoreType.DMA,
        ],
    )
    def k(list_hbm, n_hbm, eid_hbm, x_hbm, zr_hbm,
          m_hbm, cnt_hbm, xs_hbm,
          acc_vm, rows_vm, gstage_vm, astage_vm, cnt_vm,
          eid_vm, full_vm, bufa_vm, bufb_vm, nall_vm, plist_vm,
          sem, sema, semb):
        c = lax.axis_index("c")
        s = lax.axis_index("s")
        w = s * NSC + c
        lo = w * SPT

        pltpu.sync_copy(zr_hbm, acc_vm)

        @pl.loop(0, ACCP, step=16)
        def _(i):
            cnt_vm[pl.ds(i, 16)] = jnp.zeros((16,), jnp.float32)

        pltpu.sync_copy(eid_hbm, eid_vm)
        pltpu.async_copy(x_hbm.at[eid_vm.at[pl.ds(lo, SPT)]],
                         rows_vm.at[pl.ds(0, SPT)], sem).wait()
        pltpu.sync_copy(rows_vm.at[pl.ds(0, SPT)], xs_hbm.at[pl.ds(lo, SPT)])

        def drain(n_items):
            nch = (n_items + (GCH - 1)) // GCH

            def gs_body(i, carry):
                gbase = i * GCH
                for kk in range(GCH // 16):
                    pos = gbase + kk * 16
                    lane = pos + lax.iota(jnp.int32, 16)
                    valid = lane < n_items
                    pk = plist_vm[pl.ds(pos, 16)]
                    gstage_vm[pl.ds(kk * 16, 16)] = jnp.where(
                        valid, lax.shift_right_logical(pk, 8), 0)
                    astage_vm[pl.ds(kk * 16, 16)] = jnp.where(
                        valid, pk & 255, ADUMMY)
                    plsc.addupdate_scatter(
                        cnt_vm, [jnp.where(valid, pk & 255, ADUMMY)],
                        jnp.ones((16,), jnp.float32), mask=valid)
                pltpu.async_copy(x_hbm.at[gstage_vm], rows_vm, sem).wait()

                @pl.loop(0, GCH)
                def _(rr):
                    aidx = astage_vm[pl.ds(rr, 16)][0]

                    @pl.loop(0, EHID, step=16)
                    def _(kk):
                        v = rows_vm[rr, pl.ds(kk, 16)]
                        plsc.addupdate(acc_vm.at[aidx, pl.ds(kk, 16)], v)

                return carry

            lax.fori_loop(0, nch, gs_body, jnp.int32(0))

        SHORT = 512
        pltpu.sync_copy(n_hbm, nall_vm)

        def start_short(v, buf, sem_):
            pltpu.async_copy(list_hbm.at[v, pl.ds(0, SHORT)], buf, sem_)

        def wait_short(buf, sem_):
            pltpu.make_async_copy(list_hbm.at[0, pl.ds(0, SHORT)],
                                  buf, sem_).wait()

        def scan_from(ref, n_v, off):
            def vec_body(jj, off2):
                j = jj * 16
                pk = ref[pl.ds(j, 16)]
                valid = (j + lax.iota(jnp.int32, 16)) < n_v
                slot = lax.shift_right_logical(pk, 4) & 511
                sl = slot - lo
                msk = valid & (sl >= 0) & (sl < SPT)
                aidx = jnp.where(msk, sl * R + (pk & 15), 0)
                repk = (lax.shift_right_logical(pk, 13) << 8) | aidx
                plsc.store_compressed(plist_vm.at[pl.ds(off2, 16)], repk,
                                      mask=msk)
                return off2 + jnp.sum(msk.astype(jnp.int32))

            return lax.fori_loop(0, (n_v + 15) // 16, vec_body, off)

        def handle(v, buf, off):
            n_v = nall_vm[v, pl.ds(0, 16)][0]

            def short_case(o):
                return scan_from(buf, n_v, o)

            def full_case(o):
                pltpu.sync_copy(list_hbm.at[v], full_vm)
                return scan_from(full_vm, n_v, o)

            off = lax.cond(n_v <= SHORT, short_case, full_case, off)

            def do_drain(o):
                drain(o)
                return jnp.int32(0)

            return lax.cond(off >= LIST_CAP, do_drain, lambda o: o, off)

        start_short(0, bufa_vm, sema)

        def pair_body(p, off):
            v0 = 2 * p
            start_short(v0 + 1, bufb_vm, semb)
            wait_short(bufa_vm, sema)
            off = handle(v0, bufa_vm, off)

            @pl.when(v0 + 2 < NWORK)
            def _():
                start_short(v0 + 2, bufa_vm, sema)

            wait_short(bufb_vm, semb)
            return handle(v0 + 1, bufb_vm, off)

        n_left = lax.fori_loop(0, NWORK // 2, pair_body, jnp.int32(0))
        drain(n_left)

        pltpu.sync_copy(acc_vm.at[pl.ds(0, AROWS)],
                        m_hbm.at[pl.ds(w * AROWS, AROWS)])
        pltpu.sync_copy(cnt_vm.at[pl.ds(0, AROWS)],
                        cnt_hbm.at[pl.ds(w * AROWS, AROWS)])

    return k(lists, counts, eid, x, zrows)


def _sc_aggregate(eid, fidx, edge_index, et, x, zrows):
    lists, counts = _sc_scan(eid, fidx, edge_index, et)
    return _sc_accumulate(lists, counts, eid, x, zrows)


def _entity_kernel(m2, cntp, xs, comp, bases, root, rgcn_bias,
                   ep1_w1, ep1_b1, ep1_w2, ep1_b2, ep2_w, ep2_b):
    def body(m_ref, c_ref, xs_ref, comp_ref, bases_ref, root_ref, b0_ref,
             w1_ref, b1_ref, w2_ref, b2_ref, w3_ref, b3_ref, out_ref):
        inv = 1.0 / jnp.maximum(c_ref[...], 1.0)
        agg = jnp.zeros((NPOS, EHID), jnp.float32)
        for r in range(R):
            wr = jnp.zeros((EHID, EHID), jnp.float32)
            for b in range(NB):
                wr = wr + comp_ref[r, b] * bases_ref[b]
            mr = m_ref[:, r * EHID:(r + 1) * EHID] * inv[:, r:r + 1]
            agg = agg + jnp.dot(mr, wr, preferred_element_type=jnp.float32)
        x = xs_ref[...]
        ent0 = (agg + jnp.dot(x, root_ref[...],
                              preferred_element_type=jnp.float32)
                + b0_ref[...] + x)
        h = jnp.maximum(
            jnp.dot(ent0, w1_ref[...], preferred_element_type=jnp.float32)
            + b1_ref[...], 0.0)
        ent1 = (jnp.dot(h, w2_ref[...], preferred_element_type=jnp.float32)
                + b2_ref[...] + ent0)
        out_ref[...] = (jnp.dot(ent1, w3_ref[...],
                                preferred_element_type=jnp.float32)
                        + b3_ref[...])

    return pl.pallas_call(
        body,
        out_shape=jax.ShapeDtypeStruct((NPOS, HID), jnp.float32),
        in_specs=[pl.BlockSpec((NPOS, R * EHID), lambda: (0, 0)),
                 pl.BlockSpec((NPOS, R), lambda: (0, 0)),
                 pl.BlockSpec((NPOS, EHID), lambda: (0, 0)),
                 pl.BlockSpec(memory_space=pltpu.SMEM),
                 pl.BlockSpec((NB, EHID, EHID), lambda: (0, 0, 0)),
                 pl.BlockSpec((EHID, EHID), lambda: (0, 0)),
                 pl.BlockSpec((EHID,), lambda: (0,)),
                 pl.BlockSpec((EHID, EHID // 2), lambda: (0, 0)),
                 pl.BlockSpec((EHID // 2,), lambda: (0,)),
                 pl.BlockSpec((EHID // 2, EHID), lambda: (0, 0)),
                 pl.BlockSpec((EHID,), lambda: (0,)),
                 pl.BlockSpec((EHID, HID), lambda: (0, 0)),
                 pl.BlockSpec((HID,), lambda: (0,))],
        out_specs=pl.BlockSpec((NPOS, HID), lambda: (0, 0)),
    )(m2, cntp, xs, comp, bases, root, rgcn_bias,
      ep1_w1, ep1_b1, ep1_w2, ep1_b2, ep2_w, ep2_b)


def _sc_gather(ent, sidx):
    mesh = plsc.VectorSubcoreMesh(**_MESH)
    per_w = NPOS // (NSC * NTILE)

    @functools.partial(
        pl.kernel,
        out_type=jax.ShapeDtypeStruct((NPOS, HID), jnp.float32),
        mesh=mesh,
        scratch_types=[
            pltpu.VMEM((per_w,), jnp.int32),
            pltpu.VMEM((per_w, HID), jnp.float32),
            pltpu.SemaphoreType.DMA,
        ],
    )
    def k(ent_hbm, sidx_hbm, out_hbm, idx_vm, rows_vm, sem):
        c = lax.axis_index("c")
        s = lax.axis_index("s")
        base = (s * NSC + c) * per_w
        pltpu.sync_copy(sidx_hbm.at[pl.ds(base, per_w)], idx_vm)
        pltpu.async_copy(ent_hbm.at[idx_vm], rows_vm, sem).wait()
        pltpu.sync_copy(rows_vm, out_hbm.at[pl.ds(base, per_w)])

    return k(ent, sidx)


def _token_kernel(tok, w1, b1, w2, b2, w3, b3):
    def body(t_ref, w1_ref, b1_ref, w2_ref, b2_ref, w3_ref, b3_ref, o_ref):
        t0 = t_ref[...]
        h = jnp.maximum(
            jnp.dot(t0, w1_ref[...], preferred_element_type=jnp.float32)
            + b1_ref[...], 0.0)
        t1 = (jnp.dot(h, w2_ref[...], preferred_element_type=jnp.float32)
              + b2_ref[...] + t0)
        o_ref[...] = (jnp.dot(t1, w3_ref[...],
                              preferred_element_type=jnp.float32)
                      + b3_ref[...])

    return pl.pallas_call(
        body,
        out_shape=jax.ShapeDtypeStruct((BATCH * TLEN, HID), jnp.float32),
    )(tok, w1, b1, w2, b2, w3, b3)


def _attn_kernel(t2, ee, ca_w, w1, b1, w2, b2):
    def body(t_ref, e_ref, ca_ref, w1_ref, b1_ref, w2_ref, b2_ref, o_ref):
        tb = t_ref[0]
        eb = e_ref[0]
        q = jnp.dot(tb, ca_ref[...], preferred_element_type=jnp.float32)
        attn = lax.dot_general(q, eb, (((1,), (1,)), ((), ())),
                               preferred_element_type=jnp.float32) / HID
        mx = jnp.max(attn, axis=0, keepdims=True)
        ex = jnp.exp(attn - mx)
        sm = ex / jnp.sum(ex, axis=0, keepdims=True)
        p0 = lax.dot_general(sm, tb, (((0,), (0,)), ((), ())),
                             preferred_element_type=jnp.float32) + eb
        h = jnp.maximum(
            jnp.dot(p0, w1_ref[...], preferred_element_type=jnp.float32)
            + b1_ref[...], 0.0)
        o_ref[0] = (jnp.dot(h, w2_ref[...], preferred_element_type=jnp.float32)
                    + b2_ref[...] + p0)

    return pl.pallas_call(
        body,
        grid=(BATCH,),
        in_specs=[
            pl.BlockSpec((1, TLEN, HID), lambda i: (i, 0, 0)),
            pl.BlockSpec((1, ELEN, HID), lambda i: (i, 0, 0)),
            pl.BlockSpec((HID, HID), lambda i: (0, 0)),
            pl.BlockSpec((HID, HID // 2), lambda i: (0, 0)),
            pl.BlockSpec((HID // 2,), lambda i: (0,)),
            pl.BlockSpec((HID // 2, HID), lambda i: (0, 0)),
            pl.BlockSpec((HID,), lambda i: (0,)),
        ],
        out_specs=pl.BlockSpec((1, ELEN, HID), lambda i: (i, 0, 0)),
        out_shape=jax.ShapeDtypeStruct((BATCH, ELEN, HID), jnp.float32),
    )(t2, ee, ca_w, w1, b1, w2, b2)


def _pp2_kernel(x, w, b):
    OUTD = NLAYER * NBLOCK * HID
    CB = 1536

    def body(x_ref, w_ref, b_ref, o_ref):
        o_ref[...] = (jnp.dot(x_ref[...].astype(jnp.bfloat16), w_ref[...],
                              preferred_element_type=jnp.float32)
                      + b_ref[...])

    return pl.pallas_call(
        body,
        grid=(OUTD // CB,),
        in_specs=[
            pl.BlockSpec((NPOS, HID), lambda j: (0, 0)),
            pl.BlockSpec((HID, CB), lambda j: (0, j)),
            pl.BlockSpec((1, CB), lambda j: (0, j)),
        ],
        out_specs=pl.BlockSpec((NPOS, CB), lambda j: (0, j)),
        out_shape=jax.ShapeDtypeStruct((NPOS, OUTD), jnp.float32),
    )(x, w.astype(jnp.bfloat16), b.reshape(1, OUTD))


def kernel(entity_ids, token_embeds, edge_index, edge_type, node_embeds,
           comp, bases, root, rgcn_bias, ep1_w1, ep1_b1, ep1_w2, ep1_b2,
           ep2_w, ep2_b, tp1_w1, tp1_b1, tp1_w2, tp1_b2, tp2_w, tp2_b,
           ca_w, pp1_w1, pp1_b1, pp1_w2, pp1_b2, pp2_w, pp2_b):
    eid = entity_ids.reshape(NPOS).astype(jnp.int32)
    ei = edge_index.astype(jnp.int32)
    et = edge_type.astype(jnp.int32)

    fidx = _first_idx_kernel(eid.reshape(NPOS, 1),
                             eid.reshape(1, NPOS)).reshape(NPOS)

    zrows = jnp.zeros((ACCP, EHID), jnp.float32)
    m, cnt, xs = _sc_aggregate(eid, fidx, ei, et, node_embeds, zrows)

    m2 = m.reshape(NPOS, R * EHID)
    cnt2 = cnt.reshape(NPOS, R)

    ent = _entity_kernel(m2, cnt2, xs, comp, bases, root, rgcn_bias,
                         ep1_w1, ep1_b1, ep1_w2, ep1_b2, ep2_w, ep2_b)

    ee = _sc_gather(ent, fidx)

    t2 = _token_kernel(token_embeds.reshape(BATCH * TLEN, HID),
                       tp1_w1, tp1_b1, tp1_w2, tp1_b2, tp2_w, tp2_b)

    presid = _attn_kernel(t2.reshape(BATCH, TLEN, HID),
                          ee.reshape(BATCH, ELEN, HID),
                          ca_w, pp1_w1, pp1_b1, pp1_w2, pp1_b2)

    pm = _pp2_kernel(presid.reshape(NPOS, HID), pp2_w, pp2_b)

    out = pm.reshape(BATCH, ELEN, NLAYER, NBLOCK, NHEAD, HID // NHEAD)
    return jnp.transpose(out, (2, 3, 0, 4, 1, 5))


# speedup vs baseline: 1.3896x; 1.3094x over previous
import dataclasses
import functools

import jax
import jax.numpy as jnp
from jax import lax
from jax.experimental import pallas as pl
from jax.experimental.pallas import tpu as pltpu
from jax.experimental.pallas import tpu_sc as plsc

N = 10000
E = 160000
R = 12
NB = 8
HID = 768
EHID = 384
BATCH = 16
ELEN = 32
TLEN = 64
NLAYER = 12
NBLOCK = 2
NHEAD = 12

NPOS = BATCH * ELEN
NSC = 2
NTILE = 16
SLOTS_PER_SC = NPOS // NSC
ROWS_PER_SC = SLOTS_PER_SC * R
ACC_ROWS = ROWS_PER_SC + 16
SM_PAD = 10240
EDGES_PER_TILE = E // NTILE
ECH = 400
GCH = 64
LIST_PAD = EDGES_PER_TILE + GCH
KPT = NPOS // NTILE
ZROWS = 16

_MESH = dict(core_axis_name="c", subcore_axis_name="s")


def _sc_params():
    cp = pltpu.CompilerParams()
    if "needs_layout_passes" in pltpu.CompilerParams.__dataclass_fields__:
        cp = dataclasses.replace(cp, needs_layout_passes=False)
    return cp


def _first_idx_kernel(eid_col, eid_row):
    def body(a_ref, b_ref, o_ref):
        eq = a_ref[...] == b_ref[...]
        jj = lax.broadcasted_iota(jnp.int32, (NPOS, NPOS), 1)
        o_ref[...] = jnp.min(jnp.where(eq, jj, NPOS), axis=1, keepdims=True)
    return pl.pallas_call(
        body,
        out_shape=jax.ShapeDtypeStruct((NPOS, 1), jnp.int32),
    )(eid_col, eid_row)


NWORK = NSC * NTILE
SPT = NPOS // NWORK
AROWS = SPT * R
ADUMMY = AROWS
ACCP = AROWS + 8
LIST_CAP = 8192
EPW = E // NWORK
EPWP = EPW + 16


def _sc_scan(eid, fidx, edge_index, et):
    mesh = plsc.VectorSubcoreMesh(**_MESH)

    @functools.partial(
        pl.kernel,
        out_type=(
            jax.ShapeDtypeStruct((NWORK, EPWP), jnp.int32),
            jax.ShapeDtypeStruct((NWORK, 16), jnp.int32),
        ),
        mesh=mesh,
        compiler_params=_sc_params(),
        scratch_types=[
            pltpu.VMEM((SM_PAD,), jnp.int32),
            pltpu.VMEM((NPOS,), jnp.int32),
            pltpu.VMEM((NPOS,), jnp.int32),
            pltpu.VMEM((EPW,), jnp.int32),
            pltpu.VMEM((EPW,), jnp.int32),
            pltpu.VMEM((EPW,), jnp.int32),
            pltpu.VMEM((EPWP,), jnp.int32),
            pltpu.VMEM((16,), jnp.int32),
        ],
    )
    def k(eid_hbm, fidx_hbm, ei_hbm, et_hbm, list_hbm, n_hbm,
          sm_vm, eid_vm, val_vm, esrc_vm, edst_vm, etyp_vm, plist_vm, n_vm):
        c = lax.axis_index("c")
        s = lax.axis_index("s")
        w = s * NSC + c

        @pl.loop(0, SM_PAD, step=16)
        def _(i):
            sm_vm[pl.ds(i, 16)] = jnp.full((16,), -1, jnp.int32)

        pltpu.sync_copy(eid_hbm, eid_vm)
        pltpu.sync_copy(fidx_hbm, val_vm)

        @pl.loop(0, NPOS, step=16)
        def _(j):
            plsc.store_scatter(sm_vm, [eid_vm[pl.ds(j, 16)]],
                               val_vm[pl.ds(j, 16)])

        eb = w * EPW
        pltpu.sync_copy(ei_hbm.at[pl.ds(eb, EPW)], esrc_vm)
        pltpu.sync_copy(ei_hbm.at[pl.ds(E + eb, EPW)], edst_vm)
        pltpu.sync_copy(et_hbm.at[pl.ds(eb, EPW)], etyp_vm)

        def vec_body(jj, off):
            j = jj * 16
            sv = esrc_vm[pl.ds(j, 16)]
            dv = edst_vm[pl.ds(j, 16)]
            tv = etyp_vm[pl.ds(j, 16)]
            slot = plsc.load_gather(sm_vm, [dv])
            msk = slot >= 0
            packed = (sv << 13) | jnp.where(msk, slot << 4, 0) | tv
            plsc.store_compressed(plist_vm.at[pl.ds(off, 16)], packed,
                                  mask=msk)
            return off + jnp.sum(msk.astype(jnp.int32))

        n_sel = lax.fori_loop(0, EPW // 16, vec_body, jnp.int32(0))

        n_vm[pl.ds(0, 16)] = jnp.where(
            lax.iota(jnp.int32, 16) == 0, n_sel, 0)
        pltpu.sync_copy(plist_vm, list_hbm.at[w])
        pltpu.sync_copy(n_vm, n_hbm.at[w])

    return k(eid, fidx, edge_index.reshape(2 * E), et)


def _sc_accumulate(lists, counts, eid, x, zrows):
    mesh = plsc.VectorSubcoreMesh(**_MESH)

    @functools.partial(
        pl.kernel,
        out_type=(
            jax.ShapeDtypeStruct((NPOS * R, EHID), jnp.float32),
            jax.ShapeDtypeStruct((NPOS * R,), jnp.float32),
            jax.ShapeDtypeStruct((NPOS, EHID), jnp.float32),
        ),
        mesh=mesh,
        compiler_params=_sc_params(),
        scratch_types=[
            pltpu.VMEM((ACCP, EHID), jnp.float32),
            pltpu.VMEM((GCH, EHID), jnp.float32),
            pltpu.VMEM((GCH,), jnp.int32),
            pltpu.VMEM((GCH + 16,), jnp.int32),
            pltpu.VMEM((ACCP,), jnp.float32),
            pltpu.VMEM((NPOS,), jnp.int32),
            pltpu.VMEM((EPWP,), jnp.int32),
            pltpu.VMEM((512,), jnp.int32),
            pltpu.VMEM((512,), jnp.int32),
            pltpu.VMEM((NWORK, 16), jnp.int32),
            pltpu.VMEM((LIST_CAP + EPWP,), jnp.int32),
            pltpu.SemaphoreType.DMA,
            pltpu.SemaphoreType.DMA,
            pltpu.SemaphoreType.DMA,
        ],
    )
    def k(list_hbm, n_hbm, eid_hbm, x_hbm, zr_hbm,
          m_hbm, cnt_hbm, xs_hbm,
          acc_vm, rows_vm, gstage_vm, astage_vm, cnt_vm,
          eid_vm, full_vm, bufa_vm, bufb_vm, nall_vm, plist_vm,
          sem, sema, semb):
        c = lax.axis_index("c")
        s = lax.axis_index("s")
        w = s * NSC + c
        lo = w * SPT

        pltpu.sync_copy(zr_hbm, acc_vm)

        @pl.loop(0, ACCP, step=16)
        def _(i):
            cnt_vm[pl.ds(i, 16)] = jnp.zeros((16,), jnp.float32)

        pltpu.sync_copy(eid_hbm, eid_vm)
        pltpu.async_copy(x_hbm.at[eid_vm.at[pl.ds(lo, SPT)]],
                         rows_vm.at[pl.ds(0, SPT)], sem).wait()
        pltpu.sync_copy(rows_vm.at[pl.ds(0, SPT)], xs_hbm.at[pl.ds(lo, SPT)])

        def drain(n_items):
            nch = (n_items + (GCH - 1)) // GCH

            def gs_body(i, carry):
                gbase = i * GCH
                for kk in range(GCH // 16):
                    pos = gbase + kk * 16
                    lane = pos + lax.iota(jnp.int32, 16)
                    valid = lane < n_items
                    pk = plist_vm[pl.ds(pos, 16)]
                    gstage_vm[pl.ds(kk * 16, 16)] = jnp.where(
                        valid, lax.shift_right_logical(pk, 8), 0)
                    astage_vm[pl.ds(kk * 16, 16)] = jnp.where(
                        valid, pk & 255, ADUMMY)
                    plsc.addupdate_scatter(
                        cnt_vm, [jnp.where(valid, pk & 255, ADUMMY)],
                        jnp.ones((16,), jnp.float32), mask=valid)
                pltpu.async_copy(x_hbm.at[gstage_vm], rows_vm, sem).wait()

                @pl.loop(0, GCH)
                def _(rr):
                    aidx = astage_vm[pl.ds(rr, 16)][0]

                    @pl.loop(0, EHID, step=16)
                    def _(kk):
                        v = rows_vm[rr, pl.ds(kk, 16)]
                        plsc.addupdate(acc_vm.at[aidx, pl.ds(kk, 16)], v)

                return carry

            lax.fori_loop(0, nch, gs_body, jnp.int32(0))

        SHORT = 512
        pltpu.sync_copy(n_hbm, nall_vm)

        def start_short(v, buf, sem_):
            pltpu.async_copy(list_hbm.at[v, pl.ds(0, SHORT)], buf, sem_)

        def wait_short(buf, sem_):
            pltpu.make_async_copy(list_hbm.at[0, pl.ds(0, SHORT)],
                                  buf, sem_).wait()

        def scan_from(ref, n_v, off):
            def vec_body(jj, off2):
                j = jj * 16
                pk = ref[pl.ds(j, 16)]
                valid = (j + lax.iota(jnp.int32, 16)) < n_v
                slot = lax.shift_right_logical(pk, 4) & 511
                sl = slot - lo
                msk = valid & (sl >= 0) & (sl < SPT)
                aidx = jnp.where(msk, sl * R + (pk & 15), 0)
                repk = (lax.shift_right_logical(pk, 13) << 8) | aidx
                plsc.store_compressed(plist_vm.at[pl.ds(off2, 16)], repk,
                                      mask=msk)
                return off2 + jnp.sum(msk.astype(jnp.int32))

            return lax.fori_loop(0, (n_v + 15) // 16, vec_body, off)

        def handle(v, buf, off):
            n_v = nall_vm[v, pl.ds(0, 16)][0]

            def short_case(o):
                return scan_from(buf, n_v, o)

            def full_case(o):
                pltpu.sync_copy(list_hbm.at[v], full_vm)
                return scan_from(full_vm, n_v, o)

            off = lax.cond(n_v <= SHORT, short_case, full_case, off)

            def do_drain(o):
                drain(o)
                return jnp.int32(0)

            return lax.cond(off >= LIST_CAP, do_drain, lambda o: o, off)

        start_short(0, bufa_vm, sema)

        def pair_body(p, off):
            v0 = 2 * p
            start_short(v0 + 1, bufb_vm, semb)
            wait_short(bufa_vm, sema)
            off = handle(v0, bufa_vm, off)

            @pl.when(v0 + 2 < NWORK)
            def _():
                start_short(v0 + 2, bufa_vm, sema)

            wait_short(bufb_vm, semb)
            return handle(v0 + 1, bufb_vm, off)

        n_left = lax.fori_loop(0, NWORK // 2, pair_body, jnp.int32(0))
        drain(n_left)

        pltpu.sync_copy(acc_vm.at[pl.ds(0, AROWS)],
                        m_hbm.at[pl.ds(w * AROWS, AROWS)])
        pltpu.sync_copy(cnt_vm.at[pl.ds(0, AROWS)],
                        cnt_hbm.at[pl.ds(w * AROWS, AROWS)])

    return k(lists, counts, eid, x, zrows)


def _sc_aggregate(eid, fidx, edge_index, et, x, zrows):
    lists, counts = _sc_scan(eid, fidx, edge_index, et)
    return _sc_accumulate(lists, counts, eid, x, zrows)


def _entity_kernel(m2, cntp, xs, comp, bases, root, rgcn_bias,
                   ep1_w1, ep1_b1, ep1_w2, ep1_b2, ep2_w, ep2_b):
    def body(m_ref, c_ref, xs_ref, comp_ref, bases_ref, root_ref, b0_ref,
             w1_ref, b1_ref, w2_ref, b2_ref, w3_ref, b3_ref, out_ref):
        inv = 1.0 / jnp.maximum(c_ref[...], 1.0)
        agg = jnp.zeros((NPOS, EHID), jnp.float32)
        for r in range(R):
            wr = jnp.zeros((EHID, EHID), jnp.float32)
            for b in range(NB):
                wr = wr + comp_ref[r, b] * bases_ref[b]
            mr = m_ref[:, r * EHID:(r + 1) * EHID] * inv[:, r:r + 1]
            agg = agg + jnp.dot(mr, wr, preferred_element_type=jnp.float32)
        x = xs_ref[...]
        ent0 = (agg + jnp.dot(x, root_ref[...],
                              preferred_element_type=jnp.float32)
                + b0_ref[...] + x)
        h = jnp.maximum(
            jnp.dot(ent0, w1_ref[...], preferred_element_type=jnp.float32)
            + b1_ref[...], 0.0)
        ent1 = (jnp.dot(h, w2_ref[...], preferred_element_type=jnp.float32)
                + b2_ref[...] + ent0)
        out_ref[...] = (jnp.dot(ent1, w3_ref[...],
                                preferred_element_type=jnp.float32)
                        + b3_ref[...])

    return pl.pallas_call(
        body,
        out_shape=jax.ShapeDtypeStruct((NPOS, HID), jnp.float32),
        in_specs=[pl.BlockSpec((NPOS, R * EHID), lambda: (0, 0)),
                 pl.BlockSpec((NPOS, R), lambda: (0, 0)),
                 pl.BlockSpec((NPOS, EHID), lambda: (0, 0)),
                 pl.BlockSpec(memory_space=pltpu.SMEM),
                 pl.BlockSpec((NB, EHID, EHID), lambda: (0, 0, 0)),
                 pl.BlockSpec((EHID, EHID), lambda: (0, 0)),
                 pl.BlockSpec((EHID,), lambda: (0,)),
                 pl.BlockSpec((EHID, EHID // 2), lambda: (0, 0)),
                 pl.BlockSpec((EHID // 2,), lambda: (0,)),
                 pl.BlockSpec((EHID // 2, EHID), lambda: (0, 0)),
                 pl.BlockSpec((EHID,), lambda: (0,)),
                 pl.BlockSpec((EHID, HID), lambda: (0, 0)),
                 pl.BlockSpec((HID,), lambda: (0,))],
        out_specs=pl.BlockSpec((NPOS, HID), lambda: (0, 0)),
    )(m2, cntp, xs, comp, bases, root, rgcn_bias,
      ep1_w1, ep1_b1, ep1_w2, ep1_b2, ep2_w, ep2_b)


def _sc_gather(ent, sidx):
    mesh = plsc.VectorSubcoreMesh(**_MESH)
    per_w = NPOS // (NSC * NTILE)

    @functools.partial(
        pl.kernel,
        out_type=jax.ShapeDtypeStruct((NPOS, HID), jnp.float32),
        mesh=mesh,
        scratch_types=[
            pltpu.VMEM((per_w,), jnp.int32),
            pltpu.VMEM((per_w, HID), jnp.float32),
            pltpu.SemaphoreType.DMA,
        ],
    )
    def k(ent_hbm, sidx_hbm, out_hbm, idx_vm, rows_vm, sem):
        c = lax.axis_index("c")
        s = lax.axis_index("s")
        base = (s * NSC + c) * per_w
        pltpu.sync_copy(sidx_hbm.at[pl.ds(base, per_w)], idx_vm)
        pltpu.async_copy(ent_hbm.at[idx_vm], rows_vm, sem).wait()
        pltpu.sync_copy(rows_vm, out_hbm.at[pl.ds(base, per_w)])

    return k(ent, sidx)


def _token_kernel(tok, w1, b1, w2, b2, w3, b3):
    def body(t_ref, w1_ref, b1_ref, w2_ref, b2_ref, w3_ref, b3_ref, o_ref):
        t0 = t_ref[...]
        h = jnp.maximum(
            jnp.dot(t0, w1_ref[...], preferred_element_type=jnp.float32)
            + b1_ref[...], 0.0)
        t1 = (jnp.dot(h, w2_ref[...], preferred_element_type=jnp.float32)
              + b2_ref[...] + t0)
        o_ref[...] = (jnp.dot(t1, w3_ref[...],
                              preferred_element_type=jnp.float32)
                      + b3_ref[...])

    return pl.pallas_call(
        body,
        out_shape=jax.ShapeDtypeStruct((BATCH * TLEN, HID), jnp.float32),
    )(tok, w1, b1, w2, b2, w3, b3)


def _attn_kernel(t2, ee, ca_w, w1, b1, w2, b2):
    def body(t_ref, e_ref, ca_ref, w1_ref, b1_ref, w2_ref, b2_ref, o_ref):
        tb = t_ref[0]
        eb = e_ref[0]
        q = jnp.dot(tb, ca_ref[...], preferred_element_type=jnp.float32)
        attn = lax.dot_general(q, eb, (((1,), (1,)), ((), ())),
                               preferred_element_type=jnp.float32) / HID
        mx = jnp.max(attn, axis=0, keepdims=True)
        ex = jnp.exp(attn - mx)
        sm = ex / jnp.sum(ex, axis=0, keepdims=True)
        p0 = lax.dot_general(sm, tb, (((0,), (0,)), ((), ())),
                             preferred_element_type=jnp.float32) + eb
        h = jnp.maximum(
            jnp.dot(p0, w1_ref[...], preferred_element_type=jnp.float32)
            + b1_ref[...], 0.0)
        o_ref[0] = (jnp.dot(h, w2_ref[...], preferred_element_type=jnp.float32)
                    + b2_ref[...] + p0)

    return pl.pallas_call(
        body,
        grid=(BATCH,),
        in_specs=[
            pl.BlockSpec((1, TLEN, HID), lambda i: (i, 0, 0)),
            pl.BlockSpec((1, ELEN, HID), lambda i: (i, 0, 0)),
            pl.BlockSpec((HID, HID), lambda i: (0, 0)),
            pl.BlockSpec((HID, HID // 2), lambda i: (0, 0)),
            pl.BlockSpec((HID // 2,), lambda i: (0,)),
            pl.BlockSpec((HID // 2, HID), lambda i: (0, 0)),
            pl.BlockSpec((HID,), lambda i: (0,)),
        ],
        out_specs=pl.BlockSpec((1, ELEN, HID), lambda i: (i, 0, 0)),
        out_shape=jax.ShapeDtypeStruct((BATCH, ELEN, HID), jnp.float32),
    )(t2, ee, ca_w, w1, b1, w2, b2)


def _pp2_kernel(x, w, b):
    OUTD = NLAYER * NBLOCK * HID
    HD = HID // NHEAD

    def body(x_ref, w_ref, b_ref, o_ref):
        y = (jnp.dot(x_ref[...].astype(jnp.bfloat16), w_ref[...],
                     preferred_element_type=jnp.float32)
             + b_ref[...])
        o_ref[0, 0] = jnp.transpose(
            y.reshape(BATCH, ELEN, NHEAD, HD), (0, 2, 1, 3))

    return pl.pallas_call(
        body,
        grid=(NLAYER * NBLOCK,),
        in_specs=[
            pl.BlockSpec((NPOS, HID), lambda j: (0, 0)),
            pl.BlockSpec((HID, HID), lambda j: (0, j)),
            pl.BlockSpec((1, HID), lambda j: (0, j)),
        ],
        out_specs=pl.BlockSpec((1, 1, BATCH, NHEAD, ELEN, HD),
                               lambda j: (j // NBLOCK, j % NBLOCK,
                                          0, 0, 0, 0)),
        out_shape=jax.ShapeDtypeStruct(
            (NLAYER, NBLOCK, BATCH, NHEAD, ELEN, HD), jnp.float32),
    )(x, w.astype(jnp.bfloat16), b.reshape(1, OUTD))


def kernel(entity_ids, token_embeds, edge_index, edge_type, node_embeds,
           comp, bases, root, rgcn_bias, ep1_w1, ep1_b1, ep1_w2, ep1_b2,
           ep2_w, ep2_b, tp1_w1, tp1_b1, tp1_w2, tp1_b2, tp2_w, tp2_b,
           ca_w, pp1_w1, pp1_b1, pp1_w2, pp1_b2, pp2_w, pp2_b):
    eid = entity_ids.reshape(NPOS).astype(jnp.int32)
    ei = edge_index.astype(jnp.int32)
    et = edge_type.astype(jnp.int32)

    fidx = _first_idx_kernel(eid.reshape(NPOS, 1),
                             eid.reshape(1, NPOS)).reshape(NPOS)

    zrows = jnp.zeros((ACCP, EHID), jnp.float32)
    m, cnt, xs = _sc_aggregate(eid, fidx, ei, et, node_embeds, zrows)

    m2 = m.reshape(NPOS, R * EHID)
    cnt2 = cnt.reshape(NPOS, R)

    ent = _entity_kernel(m2, cnt2, xs, comp, bases, root, rgcn_bias,
                         ep1_w1, ep1_b1, ep1_w2, ep1_b2, ep2_w, ep2_b)

    ee = _sc_gather(ent, fidx)

    t2 = _token_kernel(token_embeds.reshape(BATCH * TLEN, HID),
                       tp1_w1, tp1_b1, tp1_w2, tp1_b2, tp2_w, tp2_b)

    presid = _attn_kernel(t2.reshape(BATCH, TLEN, HID),
                          ee.reshape(BATCH, ELEN, HID),
                          ca_w, pp1_w1, pp1_b1, pp1_w2, pp1_b2)

    return _pp2_kernel(presid.reshape(NPOS, HID), pp2_w, pp2_b)
